# baseline (speedup 1.0000x reference)
import jax
import jax.numpy as jnp
from jax import lax
from jax.experimental import pallas as pl
from jax.experimental.pallas import tpu as pltpu

N = 8192
B = 16
D_FEAT = 256
D_OUT = 256
K = 8

RB = 128
CB = 512
SB = 1024


def _segstats_body(pos_ref, batch_ref, out_ref):
    i = pl.program_id(0)

    @pl.when(i == 0)
    def _():
        out_ref[:] = jnp.zeros_like(out_ref)

    bv = batch_ref[0, 0, :]
    onehot = (lax.broadcasted_iota(jnp.int32, (B, SB), 0) == bv[None, :]).astype(
        jnp.float32
    )
    pos_ext = jnp.concatenate(
        [pos_ref[:], jnp.ones((SB, 1), jnp.float32)], axis=1
    )
    out_ref[:] += lax.dot_general(
        onehot, pos_ext, (((1,), (0,)), ((), ())),
        preferred_element_type=jnp.float32,
    )


def _seg_stats(pos, batch3d):
    return pl.pallas_call(
        _segstats_body,
        grid=(N // SB,),
        in_specs=[
            pl.BlockSpec((SB, 3), lambda i: (i, 0)),
            pl.BlockSpec((1, 1, SB), lambda i: (i, 0, 0)),
        ],
        out_specs=pl.BlockSpec((B, 4), lambda i: (0, 0)),
        out_shape=jax.ShapeDtypeStruct((B, 4), jnp.float32),
    )(pos, batch3d)


def _knn_body(win_ref, pos_r_ref, pos_c_ref, br_ref, bc_ref, mean_ref,
              msg_ref, bd_ref, bg_ref):
    i = pl.program_id(0)
    j = pl.program_id(1)
    nj = pl.num_programs(1)

    @pl.when(j == 0)
    def _():
        bd_ref[:] = jnp.full((RB, K), jnp.inf, jnp.float32)
        bg_ref[:] = jnp.zeros((RB, K), jnp.float32)

    def _onehot(bcol, nrows):
        return (bcol == lax.broadcasted_iota(jnp.int32, (nrows, B), 1)).astype(
            jnp.float32
        )

    active = j <= win_ref[i, 1] - win_ref[i, 0]

    @pl.when(active)
    def _tile():
        oh_r = _onehot(br_ref[:], RB)
        oh_c = _onehot(bc_ref[:], CB)
        z_r = pos_r_ref[:] - lax.dot_general(
            oh_r, mean_ref[:], (((1,), (0,)), ((), ())),
            preferred_element_type=jnp.float32,
        )
        z_c = pos_c_ref[:] - lax.dot_general(
            oh_c, mean_ref[:], (((1,), (0,)), ((), ())),
            preferred_element_type=jnp.float32,
        )
        sq_r = jnp.sum(z_r * z_r, axis=1, keepdims=True)
        sq_c = jnp.sum(z_c * z_c, axis=1, keepdims=True)
        g = lax.dot_general(
            z_r, z_c, (((1,), (1,)), ((), ())),
            preferred_element_type=jnp.float32,
        )
        a_ext = jnp.concatenate([-2.0 * z_r, jnp.ones((RB, 1), jnp.float32)], axis=1)
        b_ext = jnp.concatenate([z_c, sq_c], axis=1)
        d0 = lax.dot_general(
            a_ext, b_ext, (((1,), (1,)), ((), ())),
            preferred_element_type=jnp.float32,
        )
        dist = sq_r + d0
        same = lax.dot_general(
            oh_r, oh_c, (((1,), (1,)), ((), ())),
            preferred_element_type=jnp.float32,
        ) > 0.5
        dist = jnp.where(same, dist, jnp.inf)

        wd = jnp.concatenate([bd_ref[:], dist], axis=1)
        wg = jnp.concatenate([bg_ref[:], g], axis=1)
        iot = lax.broadcasted_iota(jnp.int32, (RB, K + CB), 1).astype(jnp.float32)
        for s in range(K):
            m = jnp.min(wd, axis=1, keepdims=True)
            fi = jnp.min(
                jnp.where(wd == m, iot, float(K + CB)), axis=1, keepdims=True
            )
            sel = iot == fi
            bd_ref[:, s : s + 1] = m
            bg_ref[:, s : s + 1] = jnp.sum(
                jnp.where(sel, wg, 0.0), axis=1, keepdims=True
            )
            wd = jnp.where(sel, jnp.inf, wd)

    @pl.when(j == nj - 1)
    def _finalize():
        oh_r = _onehot(br_ref[:], RB)
        z_r = pos_r_ref[:] - lax.dot_general(
            oh_r, mean_ref[:], (((1,), (0,)), ((), ())),
            preferred_element_type=jnp.float32,
        )
        sq_r = jnp.sum(z_r * z_r, axis=1, keepdims=True)
        msg_norm = jnp.sqrt(sq_r)
        d_k = bd_ref[:]
        g_k = bg_ref[:]
        sq_c_k = d_k - sq_r + 2.0 * g_k
        cn_k = jnp.sqrt(jnp.maximum(sq_r * sq_c_k - g_k * g_k, 0.0))
        cn_k = jnp.where(d_k < jnp.inf, cn_k, 0.0)
        cn_mean = jnp.sum(cn_k, axis=1, keepdims=True) * (1.0 / K)
        dot_mean = jnp.sum(g_k, axis=1, keepdims=True) * (1.0 / K)
        msg_ref[:] = jnp.concatenate([msg_norm, cn_mean, dot_mean], axis=1)


def _knn_msg(pos, batch_col, seg_mean, win):
    nj = N // CB

    def _cmap(i, j, w):
        return (jnp.minimum(w[i, 0] + j, w[i, 1]), 0)

    grid_spec = pltpu.PrefetchScalarGridSpec(
        num_scalar_prefetch=1,
        grid=(N // RB, nj),
        in_specs=[
            pl.BlockSpec((RB, 3), lambda i, j, w: (i, 0)),
            pl.BlockSpec((CB, 3), _cmap),
            pl.BlockSpec((RB, 1), lambda i, j, w: (i, 0)),
            pl.BlockSpec((CB, 1), _cmap),
            pl.BlockSpec((B, 3), lambda i, j, w: (0, 0)),
        ],
        out_specs=pl.BlockSpec((RB, 3), lambda i, j, w: (i, 0)),
        scratch_shapes=[
            pltpu.VMEM((RB, K), jnp.float32),
            pltpu.VMEM((RB, K), jnp.float32),
        ],
    )
    return pl.pallas_call(
        _knn_body,
        grid_spec=grid_spec,
        out_shape=jax.ShapeDtypeStruct((N, 3), jnp.float32),
    )(win, pos, pos, batch_col, batch_col, seg_mean)


def _mlp_pool_body(x_ref, msg_ref, batch_ref, w1_ref, w2_ref, b_ref, out_ref):
    i = pl.program_id(0)
    ni = pl.num_programs(0)

    @pl.when(i == 0)
    def _():
        out_ref[:] = jnp.full((B, D_OUT), -jnp.inf, jnp.float32)

    h = lax.dot_general(
        x_ref[:], w1_ref[:], (((1,), (1,)), ((), ())),
        preferred_element_type=jnp.float32,
    )
    h += lax.dot_general(
        msg_ref[:], w2_ref[:], (((1,), (1,)), ((), ())),
        preferred_element_type=jnp.float32,
    )
    h = jnp.maximum(h + b_ref[:], 0.0)
    bcol = batch_ref[:]
    for bb in range(B):
        hm = jnp.where(bcol == bb, h, -jnp.inf)
        colmax = jnp.max(hm, axis=0, keepdims=True)
        out_ref[bb : bb + 1, :] = jnp.maximum(out_ref[bb : bb + 1, :], colmax)

    @pl.when(i == ni - 1)
    def _():
        out_ref[:] = jnp.where(out_ref[:] == -jnp.inf, 0.0, out_ref[:])


def _mlp_pool(x, msg, batch_col, W, b):
    w1 = W[:, :D_FEAT]
    w2 = W[:, D_FEAT:]
    return pl.pallas_call(
        _mlp_pool_body,
        grid=(N // SB,),
        in_specs=[
            pl.BlockSpec((SB, D_FEAT), lambda i: (i, 0)),
            pl.BlockSpec((SB, 3), lambda i: (i, 0)),
            pl.BlockSpec((SB, 1), lambda i: (i, 0)),
            pl.BlockSpec((D_OUT, D_FEAT), lambda i: (0, 0)),
            pl.BlockSpec((D_OUT, 3), lambda i: (0, 0)),
            pl.BlockSpec((1, D_OUT), lambda i: (0, 0)),
        ],
        out_specs=pl.BlockSpec((B, D_OUT), lambda i: (0, 0)),
        out_shape=jax.ShapeDtypeStruct((B, D_OUT), jnp.float32),
    )(x, msg, batch_col, w1, w2, b.reshape(1, D_OUT))


def kernel(x, pos, batch, W, b):
    batch = batch.astype(jnp.int32)
    batch_col = batch.reshape(N, 1)
    batch3d_s = batch.reshape(N // SB, 1, SB)

    stats = _seg_stats(pos, batch3d_s)
    counts = stats[:, 3]
    seg_mean = stats[:, :3] / jnp.maximum(counts, 1.0)[:, None]

    starts = jnp.searchsorted(batch, jnp.arange(B, dtype=jnp.int32), side="left")
    ends = jnp.searchsorted(batch, jnp.arange(B, dtype=jnp.int32), side="right")
    first_cb = starts[batch[::RB]] // CB
    last_cb = (ends[batch[RB - 1 :: RB]] - 1) // CB
    win = jnp.stack([first_cb, last_cb], axis=1).astype(jnp.int32)

    msg = _knn_msg(pos, batch_col, seg_mean, win)
    pooled = _mlp_pool(x, msg, batch_col, W, b)

    new_pos = jnp.zeros((B, 3), dtype=pos.dtype)
    new_batch = jnp.arange(B, dtype=jnp.int64)
    return pooled, new_pos, new_batch


# speedup vs baseline: 1.1455x; 1.1455x over previous
import jax
import jax.numpy as jnp
from jax import lax
from jax.experimental import pallas as pl
from jax.experimental.pallas import tpu as pltpu

N = 8192
B = 16
D_FEAT = 256
D_OUT = 256
K = 8

RB = 256
CB = 256
SB = 1024


def _segstats_body(pos_ref, batch_ref, out_ref):
    i = pl.program_id(0)

    @pl.when(i == 0)
    def _():
        out_ref[:] = jnp.zeros_like(out_ref)

    bv = batch_ref[0, 0, :]
    onehot = (lax.broadcasted_iota(jnp.int32, (B, SB), 0) == bv[None, :]).astype(
        jnp.float32
    )
    pos_ext = jnp.concatenate(
        [pos_ref[:], jnp.ones((SB, 1), jnp.float32)], axis=1
    )
    out_ref[:] += lax.dot_general(
        onehot, pos_ext, (((1,), (0,)), ((), ())),
        preferred_element_type=jnp.float32,
    )


def _seg_stats(pos, batch3d):
    return pl.pallas_call(
        _segstats_body,
        grid=(N // SB,),
        in_specs=[
            pl.BlockSpec((SB, 3), lambda i: (i, 0)),
            pl.BlockSpec((1, 1, SB), lambda i: (i, 0, 0)),
        ],
        out_specs=pl.BlockSpec((B, 4), lambda i: (0, 0)),
        out_shape=jax.ShapeDtypeStruct((B, 4), jnp.float32),
    )(pos, batch3d)


def _knn_body(win_ref, pos_r_ref, pos_c_ref, br_ref, bc_ref, mean_ref,
              msg_ref, bd_ref, bg_ref):
    i = pl.program_id(0)
    j = pl.program_id(1)
    nj = pl.num_programs(1)

    @pl.when(j == 0)
    def _():
        bd_ref[:] = jnp.full((RB, K), jnp.inf, jnp.float32)
        bg_ref[:] = jnp.zeros((RB, K), jnp.float32)

    def _onehot(bcol, nrows):
        return (bcol == lax.broadcasted_iota(jnp.int32, (nrows, B), 1)).astype(
            jnp.float32
        )

    active = j <= win_ref[i, 1] - win_ref[i, 0]

    @pl.when(active)
    def _tile():
        oh_r = _onehot(br_ref[:], RB)
        oh_c = _onehot(bc_ref[:], CB)
        z_r = pos_r_ref[:] - lax.dot_general(
            oh_r, mean_ref[:], (((1,), (0,)), ((), ())),
            preferred_element_type=jnp.float32,
        )
        z_c = pos_c_ref[:] - lax.dot_general(
            oh_c, mean_ref[:], (((1,), (0,)), ((), ())),
            preferred_element_type=jnp.float32,
        )
        sq_r = jnp.sum(z_r * z_r, axis=1, keepdims=True)
        sq_c = jnp.sum(z_c * z_c, axis=1, keepdims=True)
        g = lax.dot_general(
            z_r, z_c, (((1,), (1,)), ((), ())),
            preferred_element_type=jnp.float32,
        )
        a_ext = jnp.concatenate([-2.0 * z_r, jnp.ones((RB, 1), jnp.float32)], axis=1)
        b_ext = jnp.concatenate([z_c, sq_c], axis=1)
        d0 = lax.dot_general(
            a_ext, b_ext, (((1,), (1,)), ((), ())),
            preferred_element_type=jnp.float32,
        )
        dist = sq_r + d0
        same = lax.dot_general(
            oh_r, oh_c, (((1,), (1,)), ((), ())),
            preferred_element_type=jnp.float32,
        ) > 0.5
        dist = jnp.where(same, dist, jnp.inf)

        wd = jnp.concatenate([bd_ref[:], dist], axis=1)
        wg = jnp.concatenate([bg_ref[:], g], axis=1)
        iot = lax.broadcasted_iota(jnp.int32, (RB, K + CB), 1).astype(jnp.float32)
        for s in range(K):
            m = jnp.min(wd, axis=1, keepdims=True)
            fi = jnp.min(
                jnp.where(wd == m, iot, float(K + CB)), axis=1, keepdims=True
            )
            sel = iot == fi
            bd_ref[:, s : s + 1] = m
            bg_ref[:, s : s + 1] = jnp.sum(
                jnp.where(sel, wg, 0.0), axis=1, keepdims=True
            )
            wd = jnp.where(sel, jnp.inf, wd)

    @pl.when(j == nj - 1)
    def _finalize():
        oh_r = _onehot(br_ref[:], RB)
        z_r = pos_r_ref[:] - lax.dot_general(
            oh_r, mean_ref[:], (((1,), (0,)), ((), ())),
            preferred_element_type=jnp.float32,
        )
        sq_r = jnp.sum(z_r * z_r, axis=1, keepdims=True)
        msg_norm = jnp.sqrt(sq_r)
        d_k = bd_ref[:]
        g_k = bg_ref[:]
        sq_c_k = d_k - sq_r + 2.0 * g_k
        cn_k = jnp.sqrt(jnp.maximum(sq_r * sq_c_k - g_k * g_k, 0.0))
        cn_k = jnp.where(d_k < jnp.inf, cn_k, 0.0)
        cn_mean = jnp.sum(cn_k, axis=1, keepdims=True) * (1.0 / K)
        dot_mean = jnp.sum(g_k, axis=1, keepdims=True) * (1.0 / K)
        msg_ref[:] = jnp.concatenate([msg_norm, cn_mean, dot_mean], axis=1)


def _knn_msg(pos, batch_col, seg_mean, win):
    nj = N // CB

    def _cmap(i, j, w):
        return (jnp.minimum(w[i, 0] + j, w[i, 1]), 0)

    grid_spec = pltpu.PrefetchScalarGridSpec(
        num_scalar_prefetch=1,
        grid=(N // RB, nj),
        in_specs=[
            pl.BlockSpec((RB, 3), lambda i, j, w: (i, 0)),
            pl.BlockSpec((CB, 3), _cmap),
            pl.BlockSpec((RB, 1), lambda i, j, w: (i, 0)),
            pl.BlockSpec((CB, 1), _cmap),
            pl.BlockSpec((B, 3), lambda i, j, w: (0, 0)),
        ],
        out_specs=pl.BlockSpec((RB, 3), lambda i, j, w: (i, 0)),
        scratch_shapes=[
            pltpu.VMEM((RB, K), jnp.float32),
            pltpu.VMEM((RB, K), jnp.float32),
        ],
    )
    return pl.pallas_call(
        _knn_body,
        grid_spec=grid_spec,
        out_shape=jax.ShapeDtypeStruct((N, 3), jnp.float32),
    )(win, pos, pos, batch_col, batch_col, seg_mean)


def _mlp_pool_body(x_ref, msg_ref, batch_ref, w1_ref, w2_ref, b_ref, out_ref):
    i = pl.program_id(0)
    ni = pl.num_programs(0)

    @pl.when(i == 0)
    def _():
        out_ref[:] = jnp.full((B, D_OUT), -jnp.inf, jnp.float32)

    h = lax.dot_general(
        x_ref[:], w1_ref[:], (((1,), (1,)), ((), ())),
        preferred_element_type=jnp.float32,
    )
    h += lax.dot_general(
        msg_ref[:], w2_ref[:], (((1,), (1,)), ((), ())),
        preferred_element_type=jnp.float32,
    )
    h = jnp.maximum(h + b_ref[:], 0.0)
    bcol = batch_ref[:]
    for bb in range(B):
        hm = jnp.where(bcol == bb, h, -jnp.inf)
        colmax = jnp.max(hm, axis=0, keepdims=True)
        out_ref[bb : bb + 1, :] = jnp.maximum(out_ref[bb : bb + 1, :], colmax)

    @pl.when(i == ni - 1)
    def _():
        out_ref[:] = jnp.where(out_ref[:] == -jnp.inf, 0.0, out_ref[:])


def _mlp_pool(x, msg, batch_col, W, b):
    w1 = W[:, :D_FEAT]
    w2 = W[:, D_FEAT:]
    return pl.pallas_call(
        _mlp_pool_body,
        grid=(N // SB,),
        in_specs=[
            pl.BlockSpec((SB, D_FEAT), lambda i: (i, 0)),
            pl.BlockSpec((SB, 3), lambda i: (i, 0)),
            pl.BlockSpec((SB, 1), lambda i: (i, 0)),
            pl.BlockSpec((D_OUT, D_FEAT), lambda i: (0, 0)),
            pl.BlockSpec((D_OUT, 3), lambda i: (0, 0)),
            pl.BlockSpec((1, D_OUT), lambda i: (0, 0)),
        ],
        out_specs=pl.BlockSpec((B, D_OUT), lambda i: (0, 0)),
        out_shape=jax.ShapeDtypeStruct((B, D_OUT), jnp.float32),
    )(x, msg, batch_col, w1, w2, b.reshape(1, D_OUT))


def kernel(x, pos, batch, W, b):
    batch = batch.astype(jnp.int32)
    batch_col = batch.reshape(N, 1)
    batch3d_s = batch.reshape(N // SB, 1, SB)

    stats = _seg_stats(pos, batch3d_s)
    counts = stats[:, 3]
    seg_mean = stats[:, :3] / jnp.maximum(counts, 1.0)[:, None]

    starts = jnp.searchsorted(batch, jnp.arange(B, dtype=jnp.int32), side="left")
    ends = jnp.searchsorted(batch, jnp.arange(B, dtype=jnp.int32), side="right")
    first_cb = starts[batch[::RB]] // CB
    last_cb = (ends[batch[RB - 1 :: RB]] - 1) // CB
    win = jnp.stack([first_cb, last_cb], axis=1).astype(jnp.int32)

    msg = _knn_msg(pos, batch_col, seg_mean, win)
    pooled = _mlp_pool(x, msg, batch_col, W, b)

    new_pos = jnp.zeros((B, 3), dtype=pos.dtype)
    new_batch = jnp.arange(B, dtype=jnp.int64)
    return pooled, new_pos, new_batch


# speedup vs baseline: 1.5851x; 1.3837x over previous
import jax
import jax.numpy as jnp
from jax import lax
from jax.experimental import pallas as pl
from jax.experimental.pallas import tpu as pltpu

N = 8192
B = 16
D_FEAT = 256
D_OUT = 256
K = 8

RB = 256
CB = 1024
SB = 1024


def _segstats_body(pos_ref, batch_ref, out_ref):
    i = pl.program_id(0)

    @pl.when(i == 0)
    def _():
        out_ref[:] = jnp.zeros_like(out_ref)

    bv = batch_ref[0, 0, :]
    onehot = (lax.broadcasted_iota(jnp.int32, (B, SB), 0) == bv[None, :]).astype(
        jnp.float32
    )
    pos_ext = jnp.concatenate(
        [pos_ref[:], jnp.ones((SB, 1), jnp.float32)], axis=1
    )
    out_ref[:] += lax.dot_general(
        onehot, pos_ext, (((1,), (0,)), ((), ())),
        preferred_element_type=jnp.float32,
    )


def _seg_stats(pos, batch3d):
    return pl.pallas_call(
        _segstats_body,
        grid=(N // SB,),
        in_specs=[
            pl.BlockSpec((SB, 3), lambda i: (i, 0)),
            pl.BlockSpec((1, 1, SB), lambda i: (i, 0, 0)),
        ],
        out_specs=pl.BlockSpec((B, 4), lambda i: (0, 0)),
        out_shape=jax.ShapeDtypeStruct((B, 4), jnp.float32),
    )(pos, batch3d)


def _knn_body(win_ref, pos_r_ref, pos_c_ref, br_ref, bc_ref, mean_ref,
              msg_ref, bd_ref, bg_ref):
    i = pl.program_id(0)
    j = pl.program_id(1)
    nj = pl.num_programs(1)

    @pl.when(j == 0)
    def _():
        bd_ref[:] = jnp.full((RB, K), jnp.inf, jnp.float32)
        bg_ref[:] = jnp.zeros((RB, K), jnp.float32)

    def _onehot(bcol, nrows):
        return (bcol == lax.broadcasted_iota(jnp.int32, (nrows, B), 1)).astype(
            jnp.float32
        )

    active = j <= win_ref[i, 1] - win_ref[i, 0]

    @pl.when(active)
    def _tile():
        oh_r = _onehot(br_ref[:], RB)
        oh_c = _onehot(bc_ref[:], CB)
        z_r = pos_r_ref[:] - lax.dot_general(
            oh_r, mean_ref[:], (((1,), (0,)), ((), ())),
            preferred_element_type=jnp.float32,
        )
        z_c = pos_c_ref[:] - lax.dot_general(
            oh_c, mean_ref[:], (((1,), (0,)), ((), ())),
            preferred_element_type=jnp.float32,
        )
        sq_r = jnp.sum(z_r * z_r, axis=1, keepdims=True)
        sq_c = jnp.sum(z_c * z_c, axis=1, keepdims=True)
        g = lax.dot_general(
            z_r, z_c, (((1,), (1,)), ((), ())),
            preferred_element_type=jnp.float32,
        )
        a_ext = jnp.concatenate([-2.0 * z_r, jnp.ones((RB, 1), jnp.float32)], axis=1)
        b_ext = jnp.concatenate([z_c, sq_c], axis=1)
        d0 = lax.dot_general(
            a_ext, b_ext, (((1,), (1,)), ((), ())),
            preferred_element_type=jnp.float32,
        )
        dist = sq_r + d0
        same = lax.dot_general(
            oh_r, oh_c, (((1,), (1,)), ((), ())),
            preferred_element_type=jnp.float32,
        ) > 0.5
        dist = jnp.where(same, dist, jnp.inf)

        wd = jnp.concatenate([bd_ref[:], dist], axis=1)
        wg = jnp.concatenate([bg_ref[:], g], axis=1)
        iot = lax.broadcasted_iota(jnp.int32, (RB, K + CB), 1).astype(jnp.float32)
        for s in range(K):
            m = jnp.min(wd, axis=1, keepdims=True)
            fi = jnp.min(
                jnp.where(wd == m, iot, float(K + CB)), axis=1, keepdims=True
            )
            sel = iot == fi
            bd_ref[:, s : s + 1] = m
            bg_ref[:, s : s + 1] = jnp.sum(
                jnp.where(sel, wg, 0.0), axis=1, keepdims=True
            )
            wd = jnp.where(sel, jnp.inf, wd)

    @pl.when(j == nj - 1)
    def _finalize():
        oh_r = _onehot(br_ref[:], RB)
        z_r = pos_r_ref[:] - lax.dot_general(
            oh_r, mean_ref[:], (((1,), (0,)), ((), ())),
            preferred_element_type=jnp.float32,
        )
        sq_r = jnp.sum(z_r * z_r, axis=1, keepdims=True)
        msg_norm = jnp.sqrt(sq_r)
        d_k = bd_ref[:]
        g_k = bg_ref[:]
        sq_c_k = d_k - sq_r + 2.0 * g_k
        cn_k = jnp.sqrt(jnp.maximum(sq_r * sq_c_k - g_k * g_k, 0.0))
        cn_k = jnp.where(d_k < jnp.inf, cn_k, 0.0)
        cn_mean = jnp.sum(cn_k, axis=1, keepdims=True) * (1.0 / K)
        dot_mean = jnp.sum(g_k, axis=1, keepdims=True) * (1.0 / K)
        msg_ref[:] = jnp.concatenate([msg_norm, cn_mean, dot_mean], axis=1)


def _knn_msg(pos, batch_col, seg_mean, win):
    nj = N // CB

    def _cmap(i, j, w):
        return (jnp.minimum(w[i, 0] + j, w[i, 1]), 0)

    grid_spec = pltpu.PrefetchScalarGridSpec(
        num_scalar_prefetch=1,
        grid=(N // RB, nj),
        in_specs=[
            pl.BlockSpec((RB, 3), lambda i, j, w: (i, 0)),
            pl.BlockSpec((CB, 3), _cmap),
            pl.BlockSpec((RB, 1), lambda i, j, w: (i, 0)),
            pl.BlockSpec((CB, 1), _cmap),
            pl.BlockSpec((B, 3), lambda i, j, w: (0, 0)),
        ],
        out_specs=pl.BlockSpec((RB, 3), lambda i, j, w: (i, 0)),
        scratch_shapes=[
            pltpu.VMEM((RB, K), jnp.float32),
            pltpu.VMEM((RB, K), jnp.float32),
        ],
    )
    return pl.pallas_call(
        _knn_body,
        grid_spec=grid_spec,
        out_shape=jax.ShapeDtypeStruct((N, 3), jnp.float32),
    )(win, pos, pos, batch_col, batch_col, seg_mean)


def _mlp_pool_body(x_ref, msg_ref, batch_ref, w1_ref, w2_ref, b_ref, out_ref):
    i = pl.program_id(0)
    ni = pl.num_programs(0)

    @pl.when(i == 0)
    def _():
        out_ref[:] = jnp.full((B, D_OUT), -jnp.inf, jnp.float32)

    h = lax.dot_general(
        x_ref[:], w1_ref[:], (((1,), (1,)), ((), ())),
        preferred_element_type=jnp.float32,
    )
    h += lax.dot_general(
        msg_ref[:], w2_ref[:], (((1,), (1,)), ((), ())),
        preferred_element_type=jnp.float32,
    )
    h = jnp.maximum(h + b_ref[:], 0.0)
    bcol = batch_ref[:]
    for bb in range(B):
        hm = jnp.where(bcol == bb, h, -jnp.inf)
        colmax = jnp.max(hm, axis=0, keepdims=True)
        out_ref[bb : bb + 1, :] = jnp.maximum(out_ref[bb : bb + 1, :], colmax)

    @pl.when(i == ni - 1)
    def _():
        out_ref[:] = jnp.where(out_ref[:] == -jnp.inf, 0.0, out_ref[:])


def _mlp_pool(x, msg, batch_col, W, b):
    w1 = W[:, :D_FEAT]
    w2 = W[:, D_FEAT:]
    return pl.pallas_call(
        _mlp_pool_body,
        grid=(N // SB,),
        in_specs=[
            pl.BlockSpec((SB, D_FEAT), lambda i: (i, 0)),
            pl.BlockSpec((SB, 3), lambda i: (i, 0)),
            pl.BlockSpec((SB, 1), lambda i: (i, 0)),
            pl.BlockSpec((D_OUT, D_FEAT), lambda i: (0, 0)),
            pl.BlockSpec((D_OUT, 3), lambda i: (0, 0)),
            pl.BlockSpec((1, D_OUT), lambda i: (0, 0)),
        ],
        out_specs=pl.BlockSpec((B, D_OUT), lambda i: (0, 0)),
        out_shape=jax.ShapeDtypeStruct((B, D_OUT), jnp.float32),
    )(x, msg, batch_col, w1, w2, b.reshape(1, D_OUT))


def kernel(x, pos, batch, W, b):
    batch = batch.astype(jnp.int32)
    batch_col = batch.reshape(N, 1)
    batch3d_s = batch.reshape(N // SB, 1, SB)

    stats = _seg_stats(pos, batch3d_s)
    counts = stats[:, 3]
    seg_mean = stats[:, :3] / jnp.maximum(counts, 1.0)[:, None]

    starts = jnp.searchsorted(batch, jnp.arange(B, dtype=jnp.int32), side="left")
    ends = jnp.searchsorted(batch, jnp.arange(B, dtype=jnp.int32), side="right")
    first_cb = starts[batch[::RB]] // CB
    last_cb = (ends[batch[RB - 1 :: RB]] - 1) // CB
    win = jnp.stack([first_cb, last_cb], axis=1).astype(jnp.int32)

    msg = _knn_msg(pos, batch_col, seg_mean, win)
    pooled = _mlp_pool(x, msg, batch_col, W, b)

    new_pos = jnp.zeros((B, 3), dtype=pos.dtype)
    new_batch = jnp.arange(B, dtype=jnp.int64)
    return pooled, new_pos, new_batch


# speedup vs baseline: 1.8512x; 1.1679x over previous
import jax
import jax.numpy as jnp
from jax import lax
from jax.experimental import pallas as pl
from jax.experimental.pallas import tpu as pltpu

N = 8192
B = 16
D_FEAT = 256
D_OUT = 256
K = 8

RB = 256
CB = 1024
SB = 1024


def _segstats_body(pos_ref, batch_ref, out_ref):
    i = pl.program_id(0)

    @pl.when(i == 0)
    def _():
        out_ref[:] = jnp.zeros_like(out_ref)

    bv = batch_ref[0, 0, :]
    onehot = (lax.broadcasted_iota(jnp.int32, (B, SB), 0) == bv[None, :]).astype(
        jnp.float32
    )
    pos_ext = jnp.concatenate(
        [pos_ref[:], jnp.ones((SB, 1), jnp.float32)], axis=1
    )
    out_ref[:] += lax.dot_general(
        onehot, pos_ext, (((1,), (0,)), ((), ())),
        preferred_element_type=jnp.float32,
    )


def _seg_stats(pos, batch3d):
    return pl.pallas_call(
        _segstats_body,
        grid=(N // SB,),
        in_specs=[
            pl.BlockSpec((SB, 3), lambda i: (i, 0)),
            pl.BlockSpec((1, 1, SB), lambda i: (i, 0, 0)),
        ],
        out_specs=pl.BlockSpec((B, 4), lambda i: (0, 0)),
        out_shape=jax.ShapeDtypeStruct((B, 4), jnp.float32),
    )(pos, batch3d)


def _knn_body(win_ref, pos_r_ref, pos_c_ref, br_ref, bc_ref, mean_ref,
              msg_ref, bd_ref, bg_ref):
    i = pl.program_id(0)
    j = pl.program_id(1)
    nj = pl.num_programs(1)

    @pl.when(j == 0)
    def _():
        bd_ref[:] = jnp.full((RB, K), jnp.inf, jnp.float32)
        bg_ref[:] = jnp.zeros((RB, K), jnp.float32)

    def _onehot(bcol, nrows):
        return (bcol == lax.broadcasted_iota(jnp.int32, (nrows, B), 1)).astype(
            jnp.float32
        )

    active = j <= win_ref[i, 1] - win_ref[i, 0]

    @pl.when(active)
    def _tile():
        oh_r = _onehot(br_ref[:], RB)
        oh_c = _onehot(bc_ref[:], CB)
        z_r = pos_r_ref[:] - lax.dot_general(
            oh_r, mean_ref[:], (((1,), (0,)), ((), ())),
            preferred_element_type=jnp.float32,
        )
        z_c = pos_c_ref[:] - lax.dot_general(
            oh_c, mean_ref[:], (((1,), (0,)), ((), ())),
            preferred_element_type=jnp.float32,
        )
        sq_r = jnp.sum(z_r * z_r, axis=1, keepdims=True)
        sq_c = jnp.sum(z_c * z_c, axis=1, keepdims=True)
        g = lax.dot_general(
            z_r, z_c, (((1,), (1,)), ((), ())),
            preferred_element_type=jnp.float32,
        )
        a_ext = jnp.concatenate([-2.0 * z_r, jnp.ones((RB, 1), jnp.float32)], axis=1)
        b_ext = jnp.concatenate([z_c, sq_c], axis=1)
        d0 = lax.dot_general(
            a_ext, b_ext, (((1,), (1,)), ((), ())),
            preferred_element_type=jnp.float32,
        )
        dist = sq_r + d0
        same = lax.dot_general(
            oh_r, oh_c, (((1,), (1,)), ((), ())),
            preferred_element_type=jnp.float32,
        ) > 0.5
        dist = jnp.where(same, dist, jnp.inf)

        wd = jnp.concatenate([bd_ref[:], dist], axis=1)
        wg = jnp.concatenate([bg_ref[:], g], axis=1)
        for s in range(K):
            m = jnp.min(wd, axis=1, keepdims=True)
            sel = wd == m
            bd_ref[:, s : s + 1] = m
            bg_ref[:, s : s + 1] = jnp.min(
                jnp.where(sel, wg, jnp.inf), axis=1, keepdims=True
            )
            wd = jnp.where(sel, jnp.inf, wd)

    @pl.when(j == nj - 1)
    def _finalize():
        oh_r = _onehot(br_ref[:], RB)
        z_r = pos_r_ref[:] - lax.dot_general(
            oh_r, mean_ref[:], (((1,), (0,)), ((), ())),
            preferred_element_type=jnp.float32,
        )
        sq_r = jnp.sum(z_r * z_r, axis=1, keepdims=True)
        msg_norm = jnp.sqrt(sq_r)
        d_k = bd_ref[:]
        g_k = jnp.where(d_k < jnp.inf, bg_ref[:], 0.0)
        sq_c_k = d_k - sq_r + 2.0 * g_k
        cn_k = jnp.sqrt(jnp.maximum(sq_r * sq_c_k - g_k * g_k, 0.0))
        cn_k = jnp.where(d_k < jnp.inf, cn_k, 0.0)
        cn_mean = jnp.sum(cn_k, axis=1, keepdims=True) * (1.0 / K)
        dot_mean = jnp.sum(g_k, axis=1, keepdims=True) * (1.0 / K)
        msg_ref[:] = jnp.concatenate([msg_norm, cn_mean, dot_mean], axis=1)


def _knn_msg(pos, batch_col, seg_mean, win):
    nj = N // CB

    def _cmap(i, j, w):
        return (jnp.minimum(w[i, 0] + j, w[i, 1]), 0)

    grid_spec = pltpu.PrefetchScalarGridSpec(
        num_scalar_prefetch=1,
        grid=(N // RB, nj),
        in_specs=[
            pl.BlockSpec((RB, 3), lambda i, j, w: (i, 0)),
            pl.BlockSpec((CB, 3), _cmap),
            pl.BlockSpec((RB, 1), lambda i, j, w: (i, 0)),
            pl.BlockSpec((CB, 1), _cmap),
            pl.BlockSpec((B, 3), lambda i, j, w: (0, 0)),
        ],
        out_specs=pl.BlockSpec((RB, 3), lambda i, j, w: (i, 0)),
        scratch_shapes=[
            pltpu.VMEM((RB, K), jnp.float32),
            pltpu.VMEM((RB, K), jnp.float32),
        ],
    )
    return pl.pallas_call(
        _knn_body,
        grid_spec=grid_spec,
        out_shape=jax.ShapeDtypeStruct((N, 3), jnp.float32),
    )(win, pos, pos, batch_col, batch_col, seg_mean)


def _mlp_pool_body(x_ref, msg_ref, batch_ref, w1_ref, w2_ref, b_ref, out_ref):
    i = pl.program_id(0)
    ni = pl.num_programs(0)

    @pl.when(i == 0)
    def _():
        out_ref[:] = jnp.full((B, D_OUT), -jnp.inf, jnp.float32)

    h = lax.dot_general(
        x_ref[:], w1_ref[:], (((1,), (1,)), ((), ())),
        preferred_element_type=jnp.float32,
    )
    h += lax.dot_general(
        msg_ref[:], w2_ref[:], (((1,), (1,)), ((), ())),
        preferred_element_type=jnp.float32,
    )
    h = jnp.maximum(h + b_ref[:], 0.0)
    bcol = batch_ref[:]
    for bb in range(B):
        hm = jnp.where(bcol == bb, h, -jnp.inf)
        colmax = jnp.max(hm, axis=0, keepdims=True)
        out_ref[bb : bb + 1, :] = jnp.maximum(out_ref[bb : bb + 1, :], colmax)

    @pl.when(i == ni - 1)
    def _():
        out_ref[:] = jnp.where(out_ref[:] == -jnp.inf, 0.0, out_ref[:])


def _mlp_pool(x, msg, batch_col, W, b):
    w1 = W[:, :D_FEAT]
    w2 = W[:, D_FEAT:]
    return pl.pallas_call(
        _mlp_pool_body,
        grid=(N // SB,),
        in_specs=[
            pl.BlockSpec((SB, D_FEAT), lambda i: (i, 0)),
            pl.BlockSpec((SB, 3), lambda i: (i, 0)),
            pl.BlockSpec((SB, 1), lambda i: (i, 0)),
            pl.BlockSpec((D_OUT, D_FEAT), lambda i: (0, 0)),
            pl.BlockSpec((D_OUT, 3), lambda i: (0, 0)),
            pl.BlockSpec((1, D_OUT), lambda i: (0, 0)),
        ],
        out_specs=pl.BlockSpec((B, D_OUT), lambda i: (0, 0)),
        out_shape=jax.ShapeDtypeStruct((B, D_OUT), jnp.float32),
    )(x, msg, batch_col, w1, w2, b.reshape(1, D_OUT))


def kernel(x, pos, batch, W, b):
    batch = batch.astype(jnp.int32)
    batch_col = batch.reshape(N, 1)
    batch3d_s = batch.reshape(N // SB, 1, SB)

    stats = _seg_stats(pos, batch3d_s)
    counts = stats[:, 3]
    seg_mean = stats[:, :3] / jnp.maximum(counts, 1.0)[:, None]

    starts = jnp.searchsorted(batch, jnp.arange(B, dtype=jnp.int32), side="left")
    ends = jnp.searchsorted(batch, jnp.arange(B, dtype=jnp.int32), side="right")
    first_cb = starts[batch[::RB]] // CB
    last_cb = (ends[batch[RB - 1 :: RB]] - 1) // CB
    win = jnp.stack([first_cb, last_cb], axis=1).astype(jnp.int32)

    msg = _knn_msg(pos, batch_col, seg_mean, win)
    pooled = _mlp_pool(x, msg, batch_col, W, b)

    new_pos = jnp.zeros((B, 3), dtype=pos.dtype)
    new_batch = jnp.arange(B, dtype=jnp.int64)
    return pooled, new_pos, new_batch
